# baseline (speedup 1.0000x reference)
import functools

import jax
import jax.numpy as jnp
from jax import lax
from jax.experimental import pallas as pl
from jax.experimental.pallas import tpu as pltpu
from jax.experimental.pallas import tpu_sc as plsc

N = 10000
NP = 10240
D = 128
H = 128
C = 16
EH = 160000
EPAD = 167936
NC = 2
NS = 16
PT = EPAD // (NC * NS)
EROWS = EPAD // 128
PROWS = 1280
NB_FAST = 64
NB_SLOW = 16
CH = 32
F32 = jnp.float32
I32 = jnp.int32

_MESH = plsc.VectorSubcoreMesh(
    core_axis_name="c", subcore_axis_name="s", num_cores=NC, num_subcores=NS)
_SC_PARAMS = pltpu.CompilerParams(needs_layout_passes=False)


def _zero16():
    return jnp.zeros((16,), F32)


@functools.partial(
    pl.kernel,
    out_type=jax.ShapeDtypeStruct((NC, NP), F32),
    mesh=_MESH,
    compiler_params=_SC_PARAMS,
    scratch_types=[
        pltpu.VMEM((PT,), I32),
        pltpu.VMEM((PT,), I32),
        pltpu.VMEM((PT,), F32),
        pltpu.VMEM((PT,), F32),
        pltpu.VMEM((NP // NS,), F32),
        pltpu.VMEM_SHARED((NP,), F32),
    ],
)
def _deg_kernel(srcf, dstf, pvf, out, src_v, dst_v, pv_v, sig_v, z_v, acc_sh):
    c = lax.axis_index("c")
    s = lax.axis_index("s")
    wid = c * NS + s
    base = wid * PT
    nslice = NP // NS

    def zbody(i, _):
        z_v[pl.ds(i * 16, 16)] = _zero16()
        return 0
    lax.fori_loop(0, nslice // 16, zbody, 0)
    pltpu.sync_copy(z_v, acc_sh.at[pl.ds(s * nslice, nslice)])
    plsc.subcore_barrier()

    pltpu.sync_copy(srcf.at[pl.ds(base, PT)], src_v)
    pltpu.sync_copy(dstf.at[pl.ds(base, PT)], dst_v)
    pltpu.sync_copy(pvf.at[pl.ds(base, PT)], pv_v)

    def body(i, _):
        sl = pl.ds(i * 16, 16)
        w = pv_v[sl]
        sig_v[sl] = 1.0 / (1.0 + jnp.exp(-w))
        return 0
    lax.fori_loop(0, PT // 16, body, 0)

    pltpu.sync_copy(sig_v, acc_sh.at[src_v], add=True)
    pltpu.sync_copy(sig_v, acc_sh.at[dst_v], add=True)
    plsc.subcore_barrier()
    pltpu.sync_copy(acc_sh.at[pl.ds(s * nslice, nslice)],
                    out.at[c, pl.ds(s * nslice, nslice)])


@functools.partial(
    pl.kernel,
    out_type=(jax.ShapeDtypeStruct((EPAD,), F32),
              jax.ShapeDtypeStruct((NC, NP), F32)),
    mesh=_MESH,
    compiler_params=_SC_PARAMS,
    scratch_types=[
        pltpu.VMEM((PT,), I32),
        pltpu.VMEM((PT,), I32),
        pltpu.VMEM((PT,), F32),
        pltpu.VMEM((PT,), F32),
        pltpu.VMEM((PT,), F32),
        pltpu.VMEM((PT,), F32),
        pltpu.VMEM((NP,), F32),
        pltpu.VMEM((NP // NS,), F32),
        pltpu.VMEM((16,), I32),
        pltpu.VMEM((16,), F32),
        pltpu.VMEM_SHARED((NP,), F32),
    ],
)
def _norm_kernel(srcf, dstf, pvf, disf, idxf, out_norm, out_w,
                 src_v, dst_v, pv_v, nrm_v, wf_v, wr_v, dis_v, z_v,
                 idx_v, sval_v, wacc_sh):
    c = lax.axis_index("c")
    s = lax.axis_index("s")
    wid = c * NS + s
    base = wid * PT
    nslice = NP // NS

    def zbody(i, _):
        z_v[pl.ds(i * 16, 16)] = _zero16()
        return 0
    lax.fori_loop(0, nslice // 16, zbody, 0)
    pltpu.sync_copy(z_v, wacc_sh.at[pl.ds(s * nslice, nslice)])
    plsc.subcore_barrier()

    pltpu.sync_copy(srcf.at[pl.ds(base, PT)], src_v)
    pltpu.sync_copy(dstf.at[pl.ds(base, PT)], dst_v)
    pltpu.sync_copy(pvf.at[pl.ds(base, PT)], pv_v)
    pltpu.sync_copy(disf, dis_v)
    pltpu.sync_copy(idxf, idx_v)

    def body(i, _):
        sl = pl.ds(i * 16, 16)
        w = pv_v[sl]
        sig = 1.0 / (1.0 + jnp.exp(-w))
        s16 = src_v[sl]
        d16 = dst_v[sl]
        dsv = plsc.load_gather(dis_v, [s16])
        ddv = plsc.load_gather(dis_v, [d16])
        nrm = dsv * sig * ddv
        nrm_v[sl] = nrm
        iv = idx_v[...]
        wf_v[sl] = jnp.where(d16 == iv, nrm, 0.0)
        wr_v[sl] = jnp.where(s16 == iv, nrm, 0.0)
        return 0
    lax.fori_loop(0, PT // 16, body, 0)

    pltpu.sync_copy(nrm_v, out_norm.at[pl.ds(base, PT)])
    pltpu.sync_copy(wf_v, wacc_sh.at[src_v], add=True)
    pltpu.sync_copy(wr_v, wacc_sh.at[dst_v], add=True)

    @pl.when(jnp.logical_and(c == 0, s == 0))
    def _():
        iv = idx_v[...]
        dv = plsc.load_gather(dis_v, [iv])
        lane = lax.iota(I32, 16)
        sval_v[...] = jnp.where(lane == 0, dv * dv, 0.0)
        pltpu.sync_copy(sval_v, wacc_sh.at[idx_v], add=True)

    plsc.subcore_barrier()
    pltpu.sync_copy(wacc_sh.at[pl.ds(s * nslice, nslice)],
                    out_w.at[c, pl.ds(s * nslice, nslice)])


@functools.partial(
    pl.kernel,
    out_type=jax.ShapeDtypeStruct((NC, NP, H), F32),
    mesh=_MESH,
    compiler_params=_SC_PARAMS,
    scratch_types=[
        pltpu.VMEM((CH, 128), I32),
        pltpu.VMEM((CH, 128), I32),
        pltpu.VMEM((CH, 128), F32),
        pltpu.VMEM((1, 128), I32),
        pltpu.VMEM((1, 128), I32),
        pltpu.VMEM((1, 128), F32),
        pltpu.VMEM((128, H), F32),
        pltpu.VMEM((128, H), F32),
        pltpu.VMEM_SHARED((NP, H), F32),
        pltpu.SemaphoreType.DMA,
        pltpu.SemaphoreType.DMA,
        pltpu.SemaphoreType.DMA,
        pltpu.SemaphoreType.DMA,
    ],
)
def _prop_kernel(m_hbm, src2d, dst2d, nrm2d, out,
                 sidx_v, didx_v, nrm_v, sidx_h, didx_h, nrm_h,
                 rb0, rb1, acc_sh, sem0, sem1, ssem0, ssem1):
    c = lax.axis_index("c")
    s = lax.axis_index("s")
    nb = jnp.where(c == 0, NB_FAST, NB_SLOW)
    rowbase = jnp.where(c == 0, s * NB_FAST, NS * NB_FAST + s * NB_SLOW)
    nslice = NP // NS
    RB = (rb0, rb1)
    SEM = (sem0, sem1)
    SSEM = (ssem0, ssem1)

    def scatter_drain(ph):
        pltpu.make_async_copy(RB[ph], acc_sh.at[sidx_h.at[0]],
                              SSEM[ph]).wait()

    def zrow(i, _):
        for j in range(H // 16):
            rb0[i, pl.ds(j * 16, 16)] = _zero16()
        return 0
    lax.fori_loop(0, 128, zrow, 0)
    for j in range(nslice // 128):
        pltpu.sync_copy(rb0, acc_sh.at[pl.ds(s * nslice + j * 128, 128)])
    plsc.subcore_barrier()

    def issue(g, ph):
        b = g // 2
        lb = b % CH

        @pl.when(jnp.logical_and(g % 2 == 0, lb == 0))
        def _():
            sl = pl.ds(pl.multiple_of(rowbase + b, 8), CH)
            pltpu.sync_copy(src2d.at[sl], sidx_v)
            pltpu.sync_copy(dst2d.at[sl], didx_v)
            pltpu.sync_copy(nrm2d.at[sl], nrm_v)

        @pl.when(g % 2 == 0)
        def _():
            pltpu.async_copy(m_hbm.at[sidx_v.at[lb]], RB[ph], SEM[ph])

        @pl.when(g % 2 == 1)
        def _():
            pltpu.async_copy(m_hbm.at[didx_v.at[lb]], RB[ph], SEM[ph])

    def wait_gather(ph):
        pltpu.make_async_copy(m_hbm.at[sidx_v.at[0]], RB[ph], SEM[ph]).wait()

    @pl.when(nb > 0)
    def _():
        issue(0, 0)

    def pair(i, _):
        for ph in range(2):
            g = i * 2 + ph
            b = g // 2
            lb = b % CH
            wait_gather(ph)

            @pl.when(g >= 1)
            def _():
                scatter_drain(1 - ph)

            @pl.when(g % 2 == 0)
            def _():
                for j in range(H // 16):
                    sl = pl.ds(j * 16, 16)
                    sidx_h[0, sl] = sidx_v[lb, sl]
                    didx_h[0, sl] = didx_v[lb, sl]
                    nrm_h[0, sl] = nrm_v[lb, sl]

            @pl.when(g + 1 < 2 * nb)
            def _():
                issue(g + 1, 1 - ph)

            def rowgrp(i16, _):
                nchunk = nrm_h[0, pl.ds(i16 * 16, 16)]
                for lane in range(16):
                    r = i16 * 16 + lane
                    nv = lax.gather(
                        nchunk, jnp.full((16, 1), lane, I32),
                        lax.GatherDimensionNumbers(
                            offset_dims=(), collapsed_slice_dims=(0,),
                            start_index_map=(0,)),
                        (1,), mode=lax.GatherScatterMode.PROMISE_IN_BOUNDS)
                    for j in range(H // 16):
                        sl = pl.ds(j * 16, 16)
                        RB[ph][r, sl] = RB[ph][r, sl] * nv
                return 0
            lax.fori_loop(0, 8, rowgrp, 0)

            @pl.when(g % 2 == 0)
            def _():
                pltpu.async_copy(RB[ph], acc_sh.at[didx_h.at[0]], SSEM[ph],
                                 add=True)

            @pl.when(g % 2 == 1)
            def _():
                pltpu.async_copy(RB[ph], acc_sh.at[sidx_h.at[0]], SSEM[ph],
                                 add=True)
        return 0
    lax.fori_loop(0, nb, pair, 0)

    @pl.when(nb > 0)
    def _():
        scatter_drain(1)

    plsc.subcore_barrier()
    for j in range(nslice // 128):
        sl = pl.ds(s * nslice + j * 128, 128)
        pltpu.sync_copy(acc_sh.at[sl], out.at[c, sl])


def _tca_body(degp_ref, xp_ref, w1_ref, m1_ref, dis_ref, sn_ref):
    deg = degp_ref[0] + degp_ref[1] + 1.0
    dis = lax.rsqrt(jnp.maximum(deg, 1e-12))
    dis_ref[...] = dis
    sn_ref[...] = dis * dis
    m1_ref[...] = jnp.dot(xp_ref[...], w1_ref[...],
                          preferred_element_type=F32)


_tca = pl.pallas_call(
    _tca_body,
    out_shape=(jax.ShapeDtypeStruct((NP, H), F32),
               jax.ShapeDtypeStruct((NP // 128, 128), F32),
               jax.ShapeDtypeStruct((NP // 128, 128), F32)),
)


def _tcb_body(acc_ref, m_ref, sn_ref, b_ref, w_ref, out_ref):
    h = acc_ref[0] + acc_ref[1] + sn_ref[...] * m_ref[...] + b_ref[...]
    h = jnp.maximum(h, 0.0)
    out_ref[...] = jnp.dot(h, w_ref[...], preferred_element_type=F32)


_tcb = pl.pallas_call(
    _tcb_body,
    out_shape=jax.ShapeDtypeStruct((NP, H), F32),
)


def _tcd_body(acc_ref, m_ref, sn_ref, b_ref, w_ref, b3_ref, wvec_ref,
              out_ref):
    h2 = acc_ref[0] + acc_ref[1] + sn_ref[...] * m_ref[...] + b_ref[...]
    h2 = jnp.maximum(h2, 0.0)
    wv = wvec_ref[...]
    wrow = wv[0:1, :] + wv[1:2, :]
    agg = jnp.dot(wrow, h2, preferred_element_type=F32)
    logits = jnp.dot(agg, w_ref[...], preferred_element_type=F32) + b3_ref[...]
    mx = jnp.max(logits, axis=-1, keepdims=True)
    sh = logits - mx
    lse = jnp.log(jnp.sum(jnp.exp(sh), axis=-1, keepdims=True))
    out_ref[...] = sh - lse


_tcd = pl.pallas_call(
    _tcd_body,
    out_shape=jax.ShapeDtypeStruct((1, C), F32),
)


def kernel(x, P_edge, P_vec, W1, b1, W2, b2, W3, b3, index):
    pad = EPAD - EH
    srcf = jnp.pad(P_edge[0].astype(I32), (0, pad))
    dstf = jnp.pad(P_edge[1].astype(I32), (0, pad))
    pvf = jnp.pad(P_vec.astype(F32), (0, pad), constant_values=-60.0)
    src2d = srcf.reshape(EROWS, 128)
    dst2d = dstf.reshape(EROWS, 128)
    idx16 = jnp.full((16,), index, dtype=I32)
    xp = jnp.pad(x.astype(F32), ((0, NP - N), (0, 0)))
    b1r = b1.reshape(1, H)
    b2r = b2.reshape(1, H)
    b3r = b3.reshape(1, C)

    deg_parts = _deg_kernel(srcf, dstf, pvf)
    m1, dis2, sn2 = _tca(deg_parts.reshape(NC, NP // 128, 128), xp, W1)
    dis_flat = dis2.reshape(NP)
    sn_col = sn2.reshape(NP, 1)

    norm_flat, w_parts = _norm_kernel(srcf, dstf, pvf, dis_flat, idx16)
    nrm2d = norm_flat.reshape(EROWS, 128)

    acc1 = _prop_kernel(m1, src2d, dst2d, nrm2d)
    m2 = _tcb(acc1, m1, sn_col, b1r, W2)
    acc2 = _prop_kernel(m2, src2d, dst2d, nrm2d)
    out = _tcd(acc2, m2, sn_col, b2r, W3, b3r, w_parts)
    return out.reshape(C)


# speedup vs baseline: 1.1057x; 1.1057x over previous
import functools

import jax
import jax.numpy as jnp
from jax import lax
from jax.experimental import pallas as pl
from jax.experimental.pallas import tpu as pltpu
from jax.experimental.pallas import tpu_sc as plsc

N = 10000
NP = 10240
D = 128
H = 128
C = 16
EH = 160000
EPAD = 167936
NC = 2
NS = 16
PT = EPAD // (NC * NS)
EROWS = EPAD // 128
PROWS = 1280
NB_FAST = 72
NB_SLOW = 8
CH = 32
F32 = jnp.float32
I32 = jnp.int32

_MESH = plsc.VectorSubcoreMesh(
    core_axis_name="c", subcore_axis_name="s", num_cores=NC, num_subcores=NS)
_SC_PARAMS = pltpu.CompilerParams(needs_layout_passes=False)


def _zero16():
    return jnp.zeros((16,), F32)


@functools.partial(
    pl.kernel,
    out_type=jax.ShapeDtypeStruct((NC, NP), F32),
    mesh=_MESH,
    compiler_params=_SC_PARAMS,
    scratch_types=[
        pltpu.VMEM((PT,), I32),
        pltpu.VMEM((PT,), I32),
        pltpu.VMEM((PT,), F32),
        pltpu.VMEM((PT,), F32),
        pltpu.VMEM((NP // NS,), F32),
        pltpu.VMEM_SHARED((NP,), F32),
    ],
)
def _deg_kernel(srcf, dstf, pvf, out, src_v, dst_v, pv_v, sig_v, z_v, acc_sh):
    c = lax.axis_index("c")
    s = lax.axis_index("s")
    wid = c * NS + s
    base = wid * PT
    nslice = NP // NS

    def zbody(i, _):
        z_v[pl.ds(i * 16, 16)] = _zero16()
        return 0
    lax.fori_loop(0, nslice // 16, zbody, 0)
    pltpu.sync_copy(z_v, acc_sh.at[pl.ds(s * nslice, nslice)])
    plsc.subcore_barrier()

    pltpu.sync_copy(srcf.at[pl.ds(base, PT)], src_v)
    pltpu.sync_copy(dstf.at[pl.ds(base, PT)], dst_v)
    pltpu.sync_copy(pvf.at[pl.ds(base, PT)], pv_v)

    def body(i, _):
        sl = pl.ds(i * 16, 16)
        w = pv_v[sl]
        sig_v[sl] = 1.0 / (1.0 + jnp.exp(-w))
        return 0
    lax.fori_loop(0, PT // 16, body, 0)

    pltpu.sync_copy(sig_v, acc_sh.at[src_v], add=True)
    pltpu.sync_copy(sig_v, acc_sh.at[dst_v], add=True)
    plsc.subcore_barrier()
    pltpu.sync_copy(acc_sh.at[pl.ds(s * nslice, nslice)],
                    out.at[c, pl.ds(s * nslice, nslice)])


@functools.partial(
    pl.kernel,
    out_type=(jax.ShapeDtypeStruct((EPAD,), F32),
              jax.ShapeDtypeStruct((NC, NP), F32)),
    mesh=_MESH,
    compiler_params=_SC_PARAMS,
    scratch_types=[
        pltpu.VMEM((PT,), I32),
        pltpu.VMEM((PT,), I32),
        pltpu.VMEM((PT,), F32),
        pltpu.VMEM((PT,), F32),
        pltpu.VMEM((PT,), F32),
        pltpu.VMEM((PT,), F32),
        pltpu.VMEM((NP,), F32),
        pltpu.VMEM((NP // NS,), F32),
        pltpu.VMEM((16,), I32),
        pltpu.VMEM((16,), F32),
        pltpu.VMEM_SHARED((NP,), F32),
    ],
)
def _norm_kernel(srcf, dstf, pvf, disf, idxf, out_norm, out_w,
                 src_v, dst_v, pv_v, nrm_v, wf_v, wr_v, dis_v, z_v,
                 idx_v, sval_v, wacc_sh):
    c = lax.axis_index("c")
    s = lax.axis_index("s")
    wid = c * NS + s
    base = wid * PT
    nslice = NP // NS

    def zbody(i, _):
        z_v[pl.ds(i * 16, 16)] = _zero16()
        return 0
    lax.fori_loop(0, nslice // 16, zbody, 0)
    pltpu.sync_copy(z_v, wacc_sh.at[pl.ds(s * nslice, nslice)])
    plsc.subcore_barrier()

    pltpu.sync_copy(srcf.at[pl.ds(base, PT)], src_v)
    pltpu.sync_copy(dstf.at[pl.ds(base, PT)], dst_v)
    pltpu.sync_copy(pvf.at[pl.ds(base, PT)], pv_v)
    pltpu.sync_copy(disf, dis_v)
    pltpu.sync_copy(idxf, idx_v)

    def body(i, _):
        sl = pl.ds(i * 16, 16)
        w = pv_v[sl]
        sig = 1.0 / (1.0 + jnp.exp(-w))
        s16 = src_v[sl]
        d16 = dst_v[sl]
        dsv = plsc.load_gather(dis_v, [s16])
        ddv = plsc.load_gather(dis_v, [d16])
        nrm = dsv * sig * ddv
        nrm_v[sl] = nrm
        iv = idx_v[...]
        wf_v[sl] = jnp.where(d16 == iv, nrm, 0.0)
        wr_v[sl] = jnp.where(s16 == iv, nrm, 0.0)
        return 0
    lax.fori_loop(0, PT // 16, body, 0)

    pltpu.sync_copy(nrm_v, out_norm.at[pl.ds(base, PT)])
    pltpu.sync_copy(wf_v, wacc_sh.at[src_v], add=True)
    pltpu.sync_copy(wr_v, wacc_sh.at[dst_v], add=True)

    @pl.when(jnp.logical_and(c == 0, s == 0))
    def _():
        iv = idx_v[...]
        dv = plsc.load_gather(dis_v, [iv])
        lane = lax.iota(I32, 16)
        sval_v[...] = jnp.where(lane == 0, dv * dv, 0.0)
        pltpu.sync_copy(sval_v, wacc_sh.at[idx_v], add=True)

    plsc.subcore_barrier()
    pltpu.sync_copy(wacc_sh.at[pl.ds(s * nslice, nslice)],
                    out_w.at[c, pl.ds(s * nslice, nslice)])


@functools.partial(
    pl.kernel,
    out_type=jax.ShapeDtypeStruct((NC, NP, H), F32),
    mesh=_MESH,
    compiler_params=_SC_PARAMS,
    scratch_types=[
        pltpu.VMEM((CH, 128), I32),
        pltpu.VMEM((CH, 128), I32),
        pltpu.VMEM((CH, 128), F32),
        pltpu.VMEM((1, 128), I32),
        pltpu.VMEM((1, 128), I32),
        pltpu.VMEM((1, 128), F32),
        pltpu.VMEM((128, H), F32),
        pltpu.VMEM((128, H), F32),
        pltpu.VMEM_SHARED((NP, H), F32),
        pltpu.SemaphoreType.DMA,
        pltpu.SemaphoreType.DMA,
        pltpu.SemaphoreType.DMA,
        pltpu.SemaphoreType.DMA,
    ],
)
def _prop_kernel(m_hbm, src2d, dst2d, nrm2d, out,
                 sidx_v, didx_v, nrm_v, sidx_h, didx_h, nrm_h,
                 rb0, rb1, acc_sh, sem0, sem1, ssem0, ssem1):
    c = lax.axis_index("c")
    s = lax.axis_index("s")
    nb = jnp.where(c == 0, NB_FAST, NB_SLOW)
    rowbase = jnp.where(c == 0, s * NB_FAST, NS * NB_FAST + s * NB_SLOW)
    nslice = NP // NS
    RB = (rb0, rb1)
    SEM = (sem0, sem1)
    SSEM = (ssem0, ssem1)

    def scatter_drain(ph):
        pltpu.make_async_copy(RB[ph], acc_sh.at[sidx_h.at[0]],
                              SSEM[ph]).wait()

    def zrow(i, _):
        for j in range(H // 16):
            rb0[i, pl.ds(j * 16, 16)] = _zero16()
        return 0
    lax.fori_loop(0, 128, zrow, 0)
    for j in range(nslice // 128):
        pltpu.sync_copy(rb0, acc_sh.at[pl.ds(s * nslice + j * 128, 128)])
    plsc.subcore_barrier()

    def issue(g, ph):
        b = g // 2
        lb = b % CH

        @pl.when(jnp.logical_and(g % 2 == 0, lb == 0))
        def _():
            sl = pl.ds(pl.multiple_of(rowbase + b, 8), CH)
            pltpu.sync_copy(src2d.at[sl], sidx_v)
            pltpu.sync_copy(dst2d.at[sl], didx_v)
            pltpu.sync_copy(nrm2d.at[sl], nrm_v)

        @pl.when(g % 2 == 0)
        def _():
            pltpu.async_copy(m_hbm.at[sidx_v.at[lb]], RB[ph], SEM[ph])

        @pl.when(g % 2 == 1)
        def _():
            pltpu.async_copy(m_hbm.at[didx_v.at[lb]], RB[ph], SEM[ph])

    def wait_gather(ph):
        pltpu.make_async_copy(m_hbm.at[sidx_v.at[0]], RB[ph], SEM[ph]).wait()

    @pl.when(nb > 0)
    def _():
        issue(0, 0)

    def pair(i, _):
        for ph in range(2):
            g = i * 2 + ph
            b = g // 2
            lb = b % CH
            wait_gather(ph)

            @pl.when(g >= 1)
            def _():
                scatter_drain(1 - ph)

            @pl.when(g % 2 == 0)
            def _():
                for j in range(H // 16):
                    sl = pl.ds(j * 16, 16)
                    sidx_h[0, sl] = sidx_v[lb, sl]
                    didx_h[0, sl] = didx_v[lb, sl]
                    nrm_h[0, sl] = nrm_v[lb, sl]

            @pl.when(g + 1 < 2 * nb)
            def _():
                issue(g + 1, 1 - ph)

            def rowgrp(i16, _):
                nchunk = nrm_h[0, pl.ds(i16 * 16, 16)]
                for lane in range(16):
                    r = i16 * 16 + lane
                    nv = lax.gather(
                        nchunk, jnp.full((16, 1), lane, I32),
                        lax.GatherDimensionNumbers(
                            offset_dims=(), collapsed_slice_dims=(0,),
                            start_index_map=(0,)),
                        (1,), mode=lax.GatherScatterMode.PROMISE_IN_BOUNDS)
                    for j in range(H // 16):
                        sl = pl.ds(j * 16, 16)
                        RB[ph][r, sl] = RB[ph][r, sl] * nv
                return 0
            lax.fori_loop(0, 8, rowgrp, 0)

            @pl.when(g % 2 == 0)
            def _():
                pltpu.async_copy(RB[ph], acc_sh.at[didx_h.at[0]], SSEM[ph],
                                 add=True)

            @pl.when(g % 2 == 1)
            def _():
                pltpu.async_copy(RB[ph], acc_sh.at[sidx_h.at[0]], SSEM[ph],
                                 add=True)
        return 0
    lax.fori_loop(0, nb, pair, 0)

    @pl.when(nb > 0)
    def _():
        scatter_drain(1)

    plsc.subcore_barrier()
    for j in range(nslice // 128):
        sl = pl.ds(s * nslice + j * 128, 128)
        pltpu.sync_copy(acc_sh.at[sl], out.at[c, sl])


def _tca_body(degp_ref, xp_ref, w1_ref, m1_ref, dis_ref, sn_ref):
    deg = degp_ref[0] + degp_ref[1] + 1.0
    dis = lax.rsqrt(jnp.maximum(deg, 1e-12))
    dis_ref[...] = dis
    sn_ref[...] = dis * dis
    m1_ref[...] = jnp.dot(xp_ref[...], w1_ref[...],
                          preferred_element_type=F32)


_tca = pl.pallas_call(
    _tca_body,
    out_shape=(jax.ShapeDtypeStruct((NP, H), F32),
               jax.ShapeDtypeStruct((NP // 128, 128), F32),
               jax.ShapeDtypeStruct((NP // 128, 128), F32)),
)


def _tcb_body(acc_ref, m_ref, sn_ref, b_ref, w_ref, out_ref):
    h = acc_ref[0] + acc_ref[1] + sn_ref[...] * m_ref[...] + b_ref[...]
    h = jnp.maximum(h, 0.0)
    out_ref[...] = jnp.dot(h, w_ref[...], preferred_element_type=F32)


_tcb = pl.pallas_call(
    _tcb_body,
    out_shape=jax.ShapeDtypeStruct((NP, H), F32),
)


def _tcd_body(acc_ref, m_ref, sn_ref, b_ref, w_ref, b3_ref, wvec_ref,
              out_ref):
    h2 = acc_ref[0] + acc_ref[1] + sn_ref[...] * m_ref[...] + b_ref[...]
    h2 = jnp.maximum(h2, 0.0)
    wv = wvec_ref[...]
    wrow = wv[0:1, :] + wv[1:2, :]
    agg = jnp.dot(wrow, h2, preferred_element_type=F32)
    logits = jnp.dot(agg, w_ref[...], preferred_element_type=F32) + b3_ref[...]
    mx = jnp.max(logits, axis=-1, keepdims=True)
    sh = logits - mx
    lse = jnp.log(jnp.sum(jnp.exp(sh), axis=-1, keepdims=True))
    out_ref[...] = sh - lse


_tcd = pl.pallas_call(
    _tcd_body,
    out_shape=jax.ShapeDtypeStruct((1, C), F32),
)


def kernel(x, P_edge, P_vec, W1, b1, W2, b2, W3, b3, index):
    pad = EPAD - EH
    srcf = jnp.pad(P_edge[0].astype(I32), (0, pad))
    dstf = jnp.pad(P_edge[1].astype(I32), (0, pad))
    pvf = jnp.pad(P_vec.astype(F32), (0, pad), constant_values=-60.0)
    src2d = srcf.reshape(EROWS, 128)
    dst2d = dstf.reshape(EROWS, 128)
    idx16 = jnp.full((16,), index, dtype=I32)
    xp = jnp.pad(x.astype(F32), ((0, NP - N), (0, 0)))
    b1r = b1.reshape(1, H)
    b2r = b2.reshape(1, H)
    b3r = b3.reshape(1, C)

    deg_parts = _deg_kernel(srcf, dstf, pvf)
    m1, dis2, sn2 = _tca(deg_parts.reshape(NC, NP // 128, 128), xp, W1)
    dis_flat = dis2.reshape(NP)
    sn_col = sn2.reshape(NP, 1)

    norm_flat, w_parts = _norm_kernel(srcf, dstf, pvf, dis_flat, idx16)
    nrm2d = norm_flat.reshape(EROWS, 128)

    acc1 = _prop_kernel(m1, src2d, dst2d, nrm2d)
    m2 = _tcb(acc1, m1, sn_col, b1r, W2)
    acc2 = _prop_kernel(m2, src2d, dst2d, nrm2d)
    out = _tcd(acc2, m2, sn_col, b2r, W3, b3r, w_parts)
    return out.reshape(C)
